# baseline (speedup 1.0000x reference)
import functools

import jax
import jax.numpy as jnp
from jax.experimental import pallas as pl

_BLK_E = 512


def _phase_a_kernel(x_ref, b1_ref, b2_ref, w0_ref, w2_ref, t_ref, n_ref):
    i = pl.program_id(0)
    xb = x_ref[...].astype(jnp.bfloat16)
    tb = jax.lax.dot_general(
        b2_ref[...].astype(jnp.bfloat16), xb, (((0,), (0,)), ((), ())),
        preferred_element_type=jnp.float32)
    nb = jnp.dot(b1_ref[...].astype(jnp.bfloat16), xb,
                 preferred_element_type=jnp.float32)

    @pl.when(i == 0)
    def _init():
        t_ref[...] = tb
        n_ref[...] = nb

    @pl.when(i > 0)
    def _acc():
        t_ref[...] += tb
        n_ref[...] += nb

    @pl.when(i == pl.num_programs(0) - 1)
    def _fold():
        t_ref[...] = jnp.dot(t_ref[...], w2_ref[...],
                             preferred_element_type=jnp.float32)
        n_ref[...] = jnp.dot(n_ref[...], w0_ref[...],
                             preferred_element_type=jnp.float32)


def _phase_b_kernel(x_ref, b1_ref, b2_ref, t_ref, n_ref, w1_ref, o_ref):
    d2 = jnp.dot(b2_ref[...].astype(jnp.bfloat16),
                 t_ref[...].astype(jnp.bfloat16),
                 preferred_element_type=jnp.float32)
    d0 = jax.lax.dot_general(
        b1_ref[...].astype(jnp.bfloat16), n_ref[...].astype(jnp.bfloat16),
        (((0,), (0,)), ((), ())),
        preferred_element_type=jnp.float32)
    d1 = jnp.dot(x_ref[...], w1_ref[...], preferred_element_type=jnp.float32)
    o_ref[...] = jnp.maximum(d2 + d1 + d0, 0.0)


@functools.partial(jax.jit, static_argnames=("interpret",))
def kernel(x, B1, B2, W0, W1, W2, interpret=False):
    n_edges, in_f = x.shape
    n_nodes = B1.shape[0]
    n_tri = B2.shape[1]
    out_f = W0.shape[1]
    grid = (n_edges // _BLK_E,)

    t, n = pl.pallas_call(
        _phase_a_kernel,
        grid=grid,
        in_specs=[
            pl.BlockSpec((_BLK_E, in_f), lambda i: (i, 0)),
            pl.BlockSpec((n_nodes, _BLK_E), lambda i: (0, i)),
            pl.BlockSpec((_BLK_E, n_tri), lambda i: (i, 0)),
            pl.BlockSpec((in_f, out_f), lambda i: (0, 0)),
            pl.BlockSpec((in_f, out_f), lambda i: (0, 0)),
        ],
        out_specs=[
            pl.BlockSpec((n_tri, out_f), lambda i: (0, 0)),
            pl.BlockSpec((n_nodes, out_f), lambda i: (0, 0)),
        ],
        out_shape=[
            jax.ShapeDtypeStruct((n_tri, out_f), jnp.float32),
            jax.ShapeDtypeStruct((n_nodes, out_f), jnp.float32),
        ],
        interpret=interpret,
    )(x, B1, B2, W0, W2)

    out = pl.pallas_call(
        _phase_b_kernel,
        grid=grid,
        in_specs=[
            pl.BlockSpec((_BLK_E, in_f), lambda i: (i, 0)),
            pl.BlockSpec((n_nodes, _BLK_E), lambda i: (0, i)),
            pl.BlockSpec((_BLK_E, n_tri), lambda i: (i, 0)),
            pl.BlockSpec((n_tri, out_f), lambda i: (0, 0)),
            pl.BlockSpec((n_nodes, out_f), lambda i: (0, 0)),
            pl.BlockSpec((in_f, out_f), lambda i: (0, 0)),
        ],
        out_specs=pl.BlockSpec((_BLK_E, out_f), lambda i: (i, 0)),
        out_shape=jax.ShapeDtypeStruct((n_edges, out_f), jnp.float32),
        interpret=interpret,
    )(x, B1, B2, t, n, W1)
    return out


# speedup vs baseline: 1.0029x; 1.0029x over previous
import functools

import jax
import jax.numpy as jnp
from jax.experimental import pallas as pl

_BLK_E = 512


def _phase_a_kernel(x_ref, b1_ref, b2_ref, w0_ref, w2_ref, t_ref, n_ref):
    i = pl.program_id(0)
    xb = x_ref[...]
    tb = jax.lax.dot_general(
        b2_ref[...], xb, (((0,), (0,)), ((), ())),
        preferred_element_type=jnp.float32)
    nb = jnp.dot(b1_ref[...], xb, preferred_element_type=jnp.float32)

    @pl.when(i == 0)
    def _init():
        t_ref[...] = tb
        n_ref[...] = nb

    @pl.when(i > 0)
    def _acc():
        t_ref[...] += tb
        n_ref[...] += nb

    @pl.when(i == pl.num_programs(0) - 1)
    def _fold():
        t_ref[...] = jnp.dot(t_ref[...], w2_ref[...],
                             preferred_element_type=jnp.float32)
        n_ref[...] = jnp.dot(n_ref[...], w0_ref[...],
                             preferred_element_type=jnp.float32)


def _phase_b_kernel(x_ref, b1_ref, b2_ref, t_ref, n_ref, w1_ref, o_ref):
    d2 = jnp.dot(b2_ref[...], t_ref[...], preferred_element_type=jnp.float32)
    d0 = jax.lax.dot_general(
        b1_ref[...], n_ref[...], (((0,), (0,)), ((), ())),
        preferred_element_type=jnp.float32)
    d1 = jnp.dot(x_ref[...], w1_ref[...], preferred_element_type=jnp.float32)
    o_ref[...] = jnp.maximum(d2 + d1 + d0, 0.0)


@functools.partial(jax.jit, static_argnames=("interpret",))
def kernel(x, B1, B2, W0, W1, W2, interpret=False):
    n_edges, in_f = x.shape
    n_nodes = B1.shape[0]
    n_tri = B2.shape[1]
    out_f = W0.shape[1]
    grid = (n_edges // _BLK_E,)

    t, n = pl.pallas_call(
        _phase_a_kernel,
        grid=grid,
        in_specs=[
            pl.BlockSpec((_BLK_E, in_f), lambda i: (i, 0)),
            pl.BlockSpec((n_nodes, _BLK_E), lambda i: (0, i)),
            pl.BlockSpec((_BLK_E, n_tri), lambda i: (i, 0)),
            pl.BlockSpec((in_f, out_f), lambda i: (0, 0)),
            pl.BlockSpec((in_f, out_f), lambda i: (0, 0)),
        ],
        out_specs=[
            pl.BlockSpec((n_tri, out_f), lambda i: (0, 0)),
            pl.BlockSpec((n_nodes, out_f), lambda i: (0, 0)),
        ],
        out_shape=[
            jax.ShapeDtypeStruct((n_tri, out_f), jnp.float32),
            jax.ShapeDtypeStruct((n_nodes, out_f), jnp.float32),
        ],
        interpret=interpret,
    )(x, B1, B2, W0, W2)

    out = pl.pallas_call(
        _phase_b_kernel,
        grid=grid,
        in_specs=[
            pl.BlockSpec((_BLK_E, in_f), lambda i: (i, 0)),
            pl.BlockSpec((n_nodes, _BLK_E), lambda i: (0, i)),
            pl.BlockSpec((_BLK_E, n_tri), lambda i: (i, 0)),
            pl.BlockSpec((n_tri, out_f), lambda i: (0, 0)),
            pl.BlockSpec((n_nodes, out_f), lambda i: (0, 0)),
            pl.BlockSpec((in_f, out_f), lambda i: (0, 0)),
        ],
        out_specs=pl.BlockSpec((_BLK_E, out_f), lambda i: (i, 0)),
        out_shape=jax.ShapeDtypeStruct((n_edges, out_f), jnp.float32),
        interpret=interpret,
    )(x, B1, B2, t, n, W1)
    return out


# speedup vs baseline: 1.0202x; 1.0173x over previous
import functools

import jax
import jax.numpy as jnp
from jax.experimental import pallas as pl
from jax.experimental.pallas import tpu as pltpu

_BLK_T = 512
_BLK_N = 512


def _tri_kernel(x_ref, b2_ref, w2_ref, d2_ref, xw2_ref):
    j = pl.program_id(0)

    @pl.when(j == 0)
    def _prep():
        xw2_ref[...] = jnp.dot(x_ref[...], w2_ref[...],
                               preferred_element_type=jnp.float32)

    b2b = b2_ref[...]
    t = jax.lax.dot_general(b2b, xw2_ref[...], (((0,), (0,)), ((), ())),
                            preferred_element_type=jnp.float32)
    d2b = jnp.dot(b2b, t, preferred_element_type=jnp.float32)

    @pl.when(j == 0)
    def _init():
        d2_ref[...] = d2b

    @pl.when(j > 0)
    def _acc():
        d2_ref[...] += d2b


def _node_kernel(x_ref, b1_ref, w0_ref, w1_ref, d2_ref, o_ref, xw0_ref):
    i = pl.program_id(0)

    @pl.when(i == 0)
    def _prep():
        xb = x_ref[...]
        xw0_ref[...] = jnp.dot(xb, w0_ref[...],
                               preferred_element_type=jnp.float32)
        o_ref[...] = d2_ref[...] + jnp.dot(xb, w1_ref[...],
                                           preferred_element_type=jnp.float32)

    b1b = b1_ref[...]
    n = jnp.dot(b1b, xw0_ref[...], preferred_element_type=jnp.float32)
    o_ref[...] += jax.lax.dot_general(b1b, n, (((0,), (0,)), ((), ())),
                                      preferred_element_type=jnp.float32)

    @pl.when(i == pl.num_programs(0) - 1)
    def _epilogue():
        o_ref[...] = jnp.maximum(o_ref[...], 0.0)


@functools.partial(jax.jit, static_argnames=("interpret",))
def kernel(x, B1, B2, W0, W1, W2, interpret=False):
    n_edges, in_f = x.shape
    n_nodes = B1.shape[0]
    n_tri = B2.shape[1]
    out_f = W0.shape[1]

    d2 = pl.pallas_call(
        _tri_kernel,
        grid=(n_tri // _BLK_T,),
        in_specs=[
            pl.BlockSpec((n_edges, in_f), lambda j: (0, 0)),
            pl.BlockSpec((n_edges, _BLK_T), lambda j: (0, j)),
            pl.BlockSpec((in_f, out_f), lambda j: (0, 0)),
        ],
        out_specs=pl.BlockSpec((n_edges, out_f), lambda j: (0, 0)),
        out_shape=jax.ShapeDtypeStruct((n_edges, out_f), jnp.float32),
        scratch_shapes=[pltpu.VMEM((n_edges, out_f), jnp.float32)],
        interpret=interpret,
    )(x, B2, W2)

    out = pl.pallas_call(
        _node_kernel,
        grid=(n_nodes // _BLK_N,),
        in_specs=[
            pl.BlockSpec((n_edges, in_f), lambda i: (0, 0)),
            pl.BlockSpec((_BLK_N, n_edges), lambda i: (i, 0)),
            pl.BlockSpec((in_f, out_f), lambda i: (0, 0)),
            pl.BlockSpec((in_f, out_f), lambda i: (0, 0)),
            pl.BlockSpec((n_edges, out_f), lambda i: (0, 0)),
        ],
        out_specs=pl.BlockSpec((n_edges, out_f), lambda i: (0, 0)),
        out_shape=jax.ShapeDtypeStruct((n_edges, out_f), jnp.float32),
        scratch_shapes=[pltpu.VMEM((n_edges, out_f), jnp.float32)],
        interpret=interpret,
    )(x, B1, W0, W1, d2)
    return out


# speedup vs baseline: 1.4403x; 1.4118x over previous
import functools

import jax
import jax.numpy as jnp
from jax.experimental import pallas as pl
from jax.experimental.pallas import tpu as pltpu

_BLK_T = 512
_BLK_N = 512


def _tri_kernel(xt_ref, b2_ref, w1_ref, w2_ref, acc_ref, xw2t_ref):
    j = pl.program_id(0)

    @pl.when(j == 0)
    def _prep():
        xt = xt_ref[...]
        xw2t_ref[...] = jax.lax.dot_general(
            w2_ref[...], xt, (((0,), (0,)), ((), ())),
            preferred_element_type=jnp.float32)
        acc_ref[...] = jax.lax.dot_general(
            xt, w1_ref[...], (((0,), (0,)), ((), ())),
            preferred_element_type=jnp.float32)

    b2b = b2_ref[...]
    tt = jnp.dot(xw2t_ref[...], b2b, preferred_element_type=jnp.float32)
    acc_ref[...] += jax.lax.dot_general(
        b2b, tt, (((1,), (1,)), ((), ())),
        preferred_element_type=jnp.float32)


def _node_kernel(xt_ref, b1_ref, w0_ref, acc_ref, o_ref, xw0_ref, d0t_ref):
    i = pl.program_id(0)

    @pl.when(i == 0)
    def _prep():
        xw0_ref[...] = jax.lax.dot_general(
            xt_ref[...], w0_ref[...], (((0,), (0,)), ((), ())),
            preferred_element_type=jnp.float32)

    b1b = b1_ref[...]
    n = jnp.dot(b1b, xw0_ref[...], preferred_element_type=jnp.float32)
    d0bt = jax.lax.dot_general(n, b1b, (((0,), (0,)), ((), ())),
                               preferred_element_type=jnp.float32)

    @pl.when(i == 0)
    def _init():
        d0t_ref[...] = d0bt

    @pl.when(i > 0)
    def _acc():
        d0t_ref[...] += d0bt

    @pl.when(i == pl.num_programs(0) - 1)
    def _epilogue():
        o_ref[...] = jnp.maximum(
            acc_ref[...] + jnp.swapaxes(d0t_ref[...], 0, 1), 0.0)


@functools.partial(jax.jit, static_argnames=("interpret",))
def kernel(x, B1, B2, W0, W1, W2, interpret=False):
    n_edges, in_f = x.shape
    n_nodes = B1.shape[0]
    n_tri = B2.shape[1]
    out_f = W0.shape[1]
    xt = jnp.swapaxes(x, 0, 1)

    acc = pl.pallas_call(
        _tri_kernel,
        grid=(n_tri // _BLK_T,),
        in_specs=[
            pl.BlockSpec((in_f, n_edges), lambda j: (0, 0)),
            pl.BlockSpec((n_edges, _BLK_T), lambda j: (0, j)),
            pl.BlockSpec((in_f, out_f), lambda j: (0, 0)),
            pl.BlockSpec((in_f, out_f), lambda j: (0, 0)),
        ],
        out_specs=pl.BlockSpec((n_edges, out_f), lambda j: (0, 0)),
        out_shape=jax.ShapeDtypeStruct((n_edges, out_f), jnp.float32),
        scratch_shapes=[pltpu.VMEM((in_f, n_edges), jnp.float32)],
        interpret=interpret,
    )(xt, B2, W1, W2)

    out = pl.pallas_call(
        _node_kernel,
        grid=(n_nodes // _BLK_N,),
        in_specs=[
            pl.BlockSpec((in_f, n_edges), lambda i: (0, 0)),
            pl.BlockSpec((_BLK_N, n_edges), lambda i: (i, 0)),
            pl.BlockSpec((in_f, out_f), lambda i: (0, 0)),
            pl.BlockSpec((n_edges, out_f), lambda i: (0, 0)),
        ],
        out_specs=pl.BlockSpec((n_edges, out_f), lambda i: (0, 0)),
        out_shape=jax.ShapeDtypeStruct((n_edges, out_f), jnp.float32),
        scratch_shapes=[pltpu.VMEM((n_edges, out_f), jnp.float32),
                        pltpu.VMEM((out_f, n_edges), jnp.float32)],
        interpret=interpret,
    )(xt, B1, W0, acc)
    return out
